# baseline (speedup 1.0000x reference)
import jax
import jax.numpy as jnp
from jax.experimental import pallas as pl
from jax.experimental.pallas import tpu as pltpu

_NUM_ACTIONS = 27
_MAX_LENGTH = 77
_T5_DIM = 4096
_NSEM = 8


def _dma_body(idx_ref, emb_hbm, out_hbm, emb_vmem, sem_t, sems):
    batch = out_hbm.shape[0]
    pltpu.make_async_copy(emb_hbm, emb_vmem, sem_t).start()
    pltpu.make_async_copy(emb_hbm, emb_vmem, sem_t).wait()

    def _copy(i, k):
        return pltpu.make_async_copy(
            emb_vmem.at[idx_ref[i]], out_hbm.at[i], sems.at[k]
        )

    for k in range(_NSEM):
        _copy(k, k).start()

    def _step(g, carry):
        for k in range(_NSEM):
            i = g * _NSEM + k
            _copy(i - _NSEM, k).wait()
            _copy(i, k).start()
        return carry

    jax.lax.fori_loop(1, batch // _NSEM, _step, 0)

    for k in range(_NSEM):
        _copy(batch - _NSEM + k, k).wait()


def kernel(indices, embeddings):
    batch = indices.shape[0]
    out = pl.pallas_call(
        _dma_body,
        grid_spec=pltpu.PrefetchScalarGridSpec(
            num_scalar_prefetch=1,
            grid=(1,),
            in_specs=[pl.BlockSpec(memory_space=pl.ANY)],
            out_specs=pl.BlockSpec(memory_space=pl.ANY),
            scratch_shapes=[
                pltpu.VMEM((_NUM_ACTIONS, _MAX_LENGTH, _T5_DIM), jnp.float32),
                pltpu.SemaphoreType.DMA,
                pltpu.SemaphoreType.DMA((_NSEM,)),
            ],
        ),
        out_shape=jax.ShapeDtypeStruct((batch, _MAX_LENGTH, _T5_DIM), jnp.float32),
    )(indices.astype(jnp.int32), embeddings)
    return out
